# baseline (speedup 1.0000x reference)
import jax
import jax.numpy as jnp
from jax.experimental import pallas as pl
from jax.experimental.pallas import tpu as pltpu

C_IN, D_OUT, N_ETAS, H_GRID, W_GRID = 512, 512, 8, 64, 128
N_COLS = H_GRID * W_GRID
TILE_N = 2048
GRID = N_COLS // TILE_N


def _fused_body(eta_ref, x_ref, bW_ref, bb_ref, eW_ref, eb_ref, out_ref, bp_ref):
    xb = x_ref[...].astype(jnp.bfloat16)
    bp = jax.lax.dot_general(xb, bW_ref[...].astype(jnp.bfloat16),
                             (((0,), (0,)), ((), ())),
                             preferred_element_type=jnp.float32)
    bp_ref[...] = bp + bb_ref[...]
    eta_b = eta_ref[0]
    acc = jnp.zeros((D_OUT, TILE_N), jnp.float32)
    for e in range(N_ETAS):
        oe = jax.lax.dot_general(eW_ref[e].astype(jnp.bfloat16), xb,
                                 (((0,), (0,)), ((), ())),
                                 preferred_element_type=jnp.float32)
        acc = jnp.where(eta_b == e, oe + eb_ref[e], acc)
    out_ref[...] = acc


def kernel(x, eta, base_W, base_b, expert_W, expert_b):
    x2 = x.reshape(C_IN, N_COLS)
    eta3 = eta.reshape(GRID, 1, TILE_N).astype(jnp.int32)
    bb2 = base_b.reshape(1, D_OUT)
    eb3 = expert_b.reshape(N_ETAS, D_OUT, 1)

    out2, bp = pl.pallas_call(
        _fused_body,
        grid=(GRID,),
        in_specs=[
            pl.BlockSpec((1, 1, TILE_N), lambda i: (i, 0, 0)),
            pl.BlockSpec((C_IN, TILE_N), lambda i: (0, i)),
            pl.BlockSpec((C_IN, D_OUT), lambda i: (0, 0)),
            pl.BlockSpec((1, D_OUT), lambda i: (0, 0)),
            pl.BlockSpec((N_ETAS, C_IN, D_OUT), lambda i: (0, 0, 0)),
            pl.BlockSpec((N_ETAS, D_OUT, 1), lambda i: (0, 0, 0)),
        ],
        out_specs=[
            pl.BlockSpec((D_OUT, TILE_N), lambda i: (0, i)),
            pl.BlockSpec((TILE_N, D_OUT), lambda i: (i, 0)),
        ],
        out_shape=[
            jax.ShapeDtypeStruct((D_OUT, N_COLS), jnp.float32),
            jax.ShapeDtypeStruct((N_COLS, D_OUT), jnp.float32),
        ],
        compiler_params=pltpu.CompilerParams(
            dimension_semantics=("parallel",)),
    )(eta3, x2, base_W, bb2, expert_W, eb3)

    return out2.reshape(D_OUT, H_GRID, W_GRID), bp
